# baseline (speedup 1.0000x reference)
import functools

import jax
import jax.numpy as jnp
from jax import lax
from jax.experimental import pallas as pl
from jax.experimental.pallas import tpu as pltpu
from jax.experimental.pallas import tpu_sc as plsc

H = 128
NUM_FRAG = 50
N_CORES = 2
N_SUBCORES = 16
NW = N_CORES * N_SUBCORES
CHUNK = 128
NBUF = 2


def _sc_embedding_sum(in_tab, out_tab, idx_in3, idx_out3, n_chunks):
    B = NW * n_chunks * CHUNK
    per_w = n_chunks * CHUNK
    mesh = plsc.VectorSubcoreMesh(core_axis_name="c", subcore_axis_name="s")

    @functools.partial(
        pl.kernel,
        mesh=mesh,
        out_type=jax.ShapeDtypeStruct((B, H), jnp.float32),
        scratch_types=[
            pltpu.VMEM_SHARED((512, H), jnp.float32),
            pltpu.VMEM_SHARED((512, H), jnp.float32),
            pltpu.VMEM((n_chunks, CHUNK), jnp.int32),
            pltpu.VMEM((n_chunks, CHUNK), jnp.int32),
        ]
        + [pltpu.VMEM((CHUNK, H), jnp.float32) for _ in range(3 * NBUF)]
        + [pltpu.SemaphoreType.DMA for _ in range(3 * NBUF)],
    )
    def k(in_tab_h, out_tab_h, ii_h, io_h, out_h, in_sp, out_sp, iiB, ioB,
          ri0, ri1, ro0, ro1, o0, o1, si0, si1, so0, so1, sw0, sw1):
        cid = lax.axis_index("c")
        sid = lax.axis_index("s")
        wid = sid * N_CORES + cid
        base = wid * per_w
        ri = [ri0, ri1]
        ro = [ro0, ro1]
        o = [o0, o1]
        si = [si0, si1]
        so = [so0, so1]
        sw = [sw0, sw1]

        @pl.when(sid == 0)
        def _():
            pltpu.sync_copy(in_tab_h, in_sp)
            pltpu.sync_copy(out_tab_h, out_sp)

        pltpu.sync_copy(ii_h.at[wid], iiB)
        pltpu.sync_copy(io_h.at[wid], ioB)
        plsc.subcore_barrier()

        def fill(b, ci):
            pltpu.async_copy(in_sp.at[iiB.at[ci]], ri[b], si[b])
            pltpu.async_copy(out_sp.at[ioB.at[ci]], ro[b], so[b])

        for b in range(NBUF):
            fill(b, b)

        def macro(m, carry):
            for b in range(NBUF):
                ci = NBUF * m + b
                off = base + ci * CHUNK
                pltpu.make_async_copy(in_sp.at[iiB.at[ci]], ri[b], si[b]).wait()
                pltpu.make_async_copy(out_sp.at[ioB.at[ci]], ro[b], so[b]).wait()

                @pl.when(m > 0)
                def _():
                    pltpu.make_async_copy(
                        o[b], out_h.at[pl.ds(base, CHUNK)], sw[b]).wait()

                @plsc.parallel_loop(0, CHUNK, unroll=2)
                def add_row(i):
                    for j in range(H // 16):
                        sl = pl.ds(j * 16, 16)
                        o[b][i, sl] = ri[b][i, sl] + ro[b][i, sl]
                pltpu.async_copy(o[b], out_h.at[pl.ds(off, CHUNK)], sw[b])

                @pl.when(ci + NBUF < n_chunks)
                def _():
                    fill(b, ci + NBUF)

            return carry

        lax.fori_loop(0, n_chunks // NBUF, macro, 0)

        for b in range(NBUF):
            pltpu.make_async_copy(
                o[b], out_h.at[pl.ds(base, CHUNK)], sw[b]).wait()

    return k(in_tab, out_tab, idx_in3, idx_out3)


def _tc_fuse(frag, emb, W, b2, tok, block_g):
    n_graph = frag.shape[0]
    grid = n_graph // block_g

    def body(frag_ref, emb_ref, w_ref, b_ref, tok_ref, out_ref):
        x = frag_ref[...].reshape(block_g * NUM_FRAG, H)
        feat = lax.dot_general(
            x, w_ref[...], (((1,), (1,)), ((), ())),
            preferred_element_type=jnp.float32,
        )
        feat = feat + emb_ref[...].reshape(block_g * NUM_FRAG, H) + b_ref[...]
        tok_rows = jnp.broadcast_to(tok_ref[...][None, :, :], (block_g, 1, H))
        out_ref[...] = jnp.concatenate(
            [tok_rows, feat.reshape(block_g, NUM_FRAG, H)], axis=1)

    return pl.pallas_call(
        body,
        grid=(grid,),
        in_specs=[
            pl.BlockSpec((block_g, NUM_FRAG, H), lambda i: (i, 0, 0)),
            pl.BlockSpec((block_g, NUM_FRAG, H), lambda i: (i, 0, 0)),
            pl.BlockSpec((H, H), lambda i: (0, 0)),
            pl.BlockSpec((1, H), lambda i: (0, 0)),
            pl.BlockSpec((1, H), lambda i: (0, 0)),
        ],
        out_specs=pl.BlockSpec((block_g, NUM_FRAG + 1, H), lambda i: (i, 0, 0)),
        out_shape=jax.ShapeDtypeStruct((n_graph, NUM_FRAG + 1, H), jnp.float32),
    )(frag, emb, W, b2, tok)


def kernel(frag_feature, in_degree, out_degree, W_feat, b_feat, in_tab, out_tab, graph_token):
    n_graph = frag_feature.shape[0]
    B = n_graph * NUM_FRAG
    n_chunks = B // (NW * CHUNK)
    idx_in3 = in_degree.reshape(NW, n_chunks, CHUNK)
    idx_out3 = out_degree.reshape(NW, n_chunks, CHUNK)
    emb = _sc_embedding_sum(in_tab, out_tab, idx_in3, idx_out3, n_chunks)
    emb = emb.reshape(n_graph, NUM_FRAG, H)
    return _tc_fuse(frag_feature, emb, W_feat, b_feat.reshape(1, H),
                    graph_token, 64)


# speedup vs baseline: 1.0065x; 1.0065x over previous
import functools

import jax
import jax.numpy as jnp
from jax import lax
from jax.experimental import pallas as pl
from jax.experimental.pallas import tpu as pltpu
from jax.experimental.pallas import tpu_sc as plsc

H = 128
NUM_FRAG = 50
N_CORES = 2
N_SUBCORES = 16
NW = N_CORES * N_SUBCORES
CHUNK = 128
NBUF = 2


def _sc_embedding_sum(in_tab, out_tab, idx_in3, idx_out3, n_chunks):
    B = NW * n_chunks * CHUNK
    per_w = n_chunks * CHUNK
    mesh = plsc.VectorSubcoreMesh(core_axis_name="c", subcore_axis_name="s")

    @functools.partial(
        pl.kernel,
        mesh=mesh,
        out_type=jax.ShapeDtypeStruct((B, H), jnp.float32),
        scratch_types=[
            pltpu.VMEM_SHARED((512, H), jnp.float32),
            pltpu.VMEM_SHARED((512, H), jnp.float32),
            pltpu.VMEM((n_chunks, CHUNK), jnp.int32),
            pltpu.VMEM((n_chunks, CHUNK), jnp.int32),
        ]
        + [pltpu.VMEM((CHUNK, H), jnp.float32) for _ in range(3 * NBUF)]
        + [pltpu.SemaphoreType.DMA for _ in range(3 * NBUF)],
    )
    def k(in_tab_h, out_tab_h, ii_h, io_h, out_h, in_sp, out_sp, iiB, ioB,
          ri0, ri1, ro0, ro1, o0, o1, si0, si1, so0, so1, sw0, sw1):
        cid = lax.axis_index("c")
        sid = lax.axis_index("s")
        wid = sid * N_CORES + cid
        base = wid * per_w
        ri = [ri0, ri1]
        ro = [ro0, ro1]
        o = [o0, o1]
        si = [si0, si1]
        so = [so0, so1]
        sw = [sw0, sw1]

        @pl.when(sid == 0)
        def _():
            pltpu.sync_copy(in_tab_h, in_sp)
            pltpu.sync_copy(out_tab_h, out_sp)

        pltpu.sync_copy(ii_h.at[wid], iiB)
        pltpu.sync_copy(io_h.at[wid], ioB)
        plsc.subcore_barrier()

        def fill(b, ci):
            pltpu.async_copy(in_sp.at[iiB.at[ci]], ri[b], si[b])
            pltpu.async_copy(out_sp.at[ioB.at[ci]], ro[b], so[b])

        for b in range(NBUF):
            fill(b, b)

        def macro(m, carry):
            for b in range(NBUF):
                ci = NBUF * m + b
                off = base + ci * CHUNK
                pltpu.make_async_copy(in_sp.at[iiB.at[ci]], ri[b], si[b]).wait()
                pltpu.make_async_copy(out_sp.at[ioB.at[ci]], ro[b], so[b]).wait()

                @pl.when(m > 0)
                def _():
                    pltpu.make_async_copy(
                        o[b], out_h.at[pl.ds(base, CHUNK)], sw[b]).wait()

                @plsc.parallel_loop(0, CHUNK, unroll=2)
                def add_row(i):
                    for j in range(H // 16):
                        sl = pl.ds(j * 16, 16)
                        o[b][i, sl] = ri[b][i, sl] + ro[b][i, sl]
                pltpu.async_copy(o[b], out_h.at[pl.ds(off, CHUNK)], sw[b])

                @pl.when(ci + NBUF < n_chunks)
                def _():
                    fill(b, ci + NBUF)

            return carry

        lax.fori_loop(0, n_chunks // NBUF, macro, 0)

        for b in range(NBUF):
            pltpu.make_async_copy(
                o[b], out_h.at[pl.ds(base, CHUNK)], sw[b]).wait()

    return k(in_tab, out_tab, idx_in3, idx_out3)


def _tc_fuse(frag, emb, W, b2, tok, block_g):
    n_graph = frag.shape[0]
    grid = n_graph // block_g

    def body(frag_ref, emb_ref, w_ref, b_ref, tok_ref, out_ref):
        x = frag_ref[...].reshape(block_g * NUM_FRAG, H)
        feat = lax.dot_general(
            x, w_ref[...], (((1,), (1,)), ((), ())),
            preferred_element_type=jnp.float32,
        )
        feat = feat + emb_ref[...].reshape(block_g * NUM_FRAG, H) + b_ref[...]
        out_ref[:, 1:, :] = feat.reshape(block_g, NUM_FRAG, H)
        out_ref[:, 0:1, :] = jnp.broadcast_to(tok_ref[...][None, :, :],
                                              (block_g, 1, H))

    return pl.pallas_call(
        body,
        grid=(grid,),
        in_specs=[
            pl.BlockSpec((block_g, NUM_FRAG, H), lambda i: (i, 0, 0)),
            pl.BlockSpec((block_g, NUM_FRAG, H), lambda i: (i, 0, 0)),
            pl.BlockSpec((H, H), lambda i: (0, 0)),
            pl.BlockSpec((1, H), lambda i: (0, 0)),
            pl.BlockSpec((1, H), lambda i: (0, 0)),
        ],
        out_specs=pl.BlockSpec((block_g, NUM_FRAG + 1, H), lambda i: (i, 0, 0)),
        out_shape=jax.ShapeDtypeStruct((n_graph, NUM_FRAG + 1, H), jnp.float32),
        compiler_params=pltpu.CompilerParams(
            dimension_semantics=("parallel",)),
    )(frag, emb, W, b2, tok)


def kernel(frag_feature, in_degree, out_degree, W_feat, b_feat, in_tab, out_tab, graph_token):
    n_graph = frag_feature.shape[0]
    B = n_graph * NUM_FRAG
    n_chunks = B // (NW * CHUNK)
    idx_in3 = in_degree.reshape(NW, n_chunks, CHUNK)
    idx_out3 = out_degree.reshape(NW, n_chunks, CHUNK)
    emb = _sc_embedding_sum(in_tab, out_tab, idx_in3, idx_out3, n_chunks)
    emb = emb.reshape(n_graph, NUM_FRAG, H)
    return _tc_fuse(frag_feature, emb, W_feat, b_feat.reshape(1, H),
                    graph_token, 64)


# speedup vs baseline: 1.3806x; 1.3716x over previous
import functools

import jax
import jax.numpy as jnp
from jax import lax
from jax.experimental import pallas as pl
from jax.experimental.pallas import tpu as pltpu
from jax.experimental.pallas import tpu_sc as plsc

H = 128
NUM_FRAG = 50
N_CORES = 2
N_SUBCORES = 16
NW = N_CORES * N_SUBCORES
CHUNK = 128
NBUF = 2


def _sc_embedding_sum(in_tab, out_tab, idx_in3, idx_out3, n_chunks):
    B = NW * n_chunks * CHUNK
    per_w = n_chunks * CHUNK
    mesh = plsc.VectorSubcoreMesh(core_axis_name="c", subcore_axis_name="s")

    @functools.partial(
        pl.kernel,
        mesh=mesh,
        out_type=jax.ShapeDtypeStruct((B, H), jnp.float32),
        scratch_types=[
            pltpu.VMEM_SHARED((512, H), jnp.float32),
            pltpu.VMEM_SHARED((512, H), jnp.float32),
            pltpu.VMEM((n_chunks, CHUNK), jnp.int32),
            pltpu.VMEM((n_chunks, CHUNK), jnp.int32),
        ]
        + [pltpu.VMEM((CHUNK, H), jnp.float32) for _ in range(3 * NBUF)]
        + [pltpu.SemaphoreType.DMA for _ in range(3 * NBUF)],
    )
    def k(in_tab_h, out_tab_h, ii_h, io_h, out_h, in_sp, out_sp, iiB, ioB,
          ri0, ri1, ro0, ro1, o0, o1, si0, si1, so0, so1, sw0, sw1):
        cid = lax.axis_index("c")
        sid = lax.axis_index("s")
        wid = sid * N_CORES + cid
        base = wid * per_w
        ri = [ri0, ri1]
        ro = [ro0, ro1]
        o = [o0, o1]
        si = [si0, si1]
        so = [so0, so1]
        sw = [sw0, sw1]

        @pl.when(sid == 0)
        def _():
            pltpu.sync_copy(in_tab_h, in_sp)
            pltpu.sync_copy(out_tab_h, out_sp)

        pltpu.sync_copy(ii_h.at[wid], iiB)
        pltpu.sync_copy(io_h.at[wid], ioB)
        plsc.subcore_barrier()

        def fill(b, ci):
            pltpu.async_copy(in_sp.at[iiB.at[ci]], ri[b], si[b])
            pltpu.async_copy(out_sp.at[ioB.at[ci]], ro[b], so[b])

        for b in range(NBUF):
            fill(b, b)

        def macro(m, carry):
            for b in range(NBUF):
                ci = NBUF * m + b
                off = base + ci * CHUNK
                pltpu.make_async_copy(in_sp.at[iiB.at[ci]], ri[b], si[b]).wait()
                pltpu.make_async_copy(out_sp.at[ioB.at[ci]], ro[b], so[b]).wait()

                @pl.when(m > 0)
                def _():
                    pltpu.make_async_copy(
                        o[b], out_h.at[pl.ds(base, CHUNK)], sw[b]).wait()

                @plsc.parallel_loop(0, CHUNK, unroll=2)
                def add_row(i):
                    for j in range(H // 16):
                        sl = pl.ds(j * 16, 16)
                        o[b][i, sl] = ri[b][i, sl] + ro[b][i, sl]
                pltpu.async_copy(o[b], out_h.at[pl.ds(off, CHUNK)], sw[b])

                @pl.when(ci + NBUF < n_chunks)
                def _():
                    fill(b, ci + NBUF)

            return carry

        lax.fori_loop(0, n_chunks // NBUF, macro, 0)

        for b in range(NBUF):
            pltpu.make_async_copy(
                o[b], out_h.at[pl.ds(base, CHUNK)], sw[b]).wait()

    return k(in_tab, out_tab, idx_in3, idx_out3)


def _tc_fuse(frag, emb, W, b2, tok, block_g):
    n_graph = frag.shape[0]
    grid = n_graph // block_g

    def body(frag_ref, emb_ref, w_ref, b_ref, tok_ref, out_ref):
        x = frag_ref[...].reshape(block_g * NUM_FRAG, H)
        feat = lax.dot_general(
            x, w_ref[...], (((1,), (1,)), ((), ())),
            preferred_element_type=jnp.float32,
        )
        feat = feat + emb_ref[...] + b_ref[...]
        out_ref[:, 1:, :] = feat.reshape(block_g, NUM_FRAG, H)
        out_ref[:, 0:1, :] = jnp.broadcast_to(tok_ref[...][None, :, :],
                                              (block_g, 1, H))

    return pl.pallas_call(
        body,
        grid=(grid,),
        in_specs=[
            pl.BlockSpec((block_g, NUM_FRAG, H), lambda i: (i, 0, 0)),
            pl.BlockSpec((block_g * NUM_FRAG, H), lambda i: (i, 0)),
            pl.BlockSpec((H, H), lambda i: (0, 0)),
            pl.BlockSpec((1, H), lambda i: (0, 0)),
            pl.BlockSpec((1, H), lambda i: (0, 0)),
        ],
        out_specs=pl.BlockSpec((block_g, NUM_FRAG + 1, H), lambda i: (i, 0, 0)),
        out_shape=jax.ShapeDtypeStruct((n_graph, NUM_FRAG + 1, H), jnp.float32),
        compiler_params=pltpu.CompilerParams(
            dimension_semantics=("parallel",)),
    )(frag, emb, W, b2, tok)


def kernel(frag_feature, in_degree, out_degree, W_feat, b_feat, in_tab, out_tab, graph_token):
    n_graph = frag_feature.shape[0]
    B = n_graph * NUM_FRAG
    n_chunks = B // (NW * CHUNK)
    idx_in3 = in_degree.reshape(NW, n_chunks, CHUNK)
    idx_out3 = out_degree.reshape(NW, n_chunks, CHUNK)
    emb = _sc_embedding_sum(in_tab, out_tab, idx_in3, idx_out3, n_chunks)
    return _tc_fuse(frag_feature, emb, W_feat, b_feat.reshape(1, H),
                    graph_token, 64)
